# baseline (speedup 1.0000x reference)
import functools

import jax
import jax.numpy as jnp
from jax import lax
from jax.experimental import pallas as pl
from jax.experimental.pallas import tpu as pltpu
from jax.experimental.pallas import tpu_sc as plsc

H = 128
NLAYERS = 6
NC, NS = 2, 16
NW = NC * NS
CH = 80
E_BLK = 2000
N_BLK = 2000
NSTAGE = 10

@functools.cache
def _mesh():
  return plsc.VectorSubcoreMesh(
      core_axis_name="c", subcore_axis_name="s", num_cores=NC, num_subcores=NS)


def _sc_gather(A, B, dst, src):
  E = dst.shape[0]
  epw = E // NW
  ch = CH if epw % CH == 0 else 40
  nch = epw // ch

  @functools.partial(
      pl.kernel,
      out_type=[jax.ShapeDtypeStruct((E, H), jnp.float32),
                jax.ShapeDtypeStruct((E, H), jnp.float32)],
      mesh=_mesh(),
      scratch_types=[pltpu.VMEM((3, ch), jnp.int32),
                     pltpu.VMEM((3, ch), jnp.int32),
                     pltpu.VMEM((3, ch, H), jnp.float32),
                     pltpu.VMEM((3, ch, H), jnp.float32),
                     pltpu.SemaphoreType.DMA,
                     pltpu.SemaphoreType.DMA,
                     pltpu.SemaphoreType.DMA],
  )
  def k(a_hbm, b_hbm, dst_hbm, src_hbm, ga_hbm, gb_hbm, di, si, ra, rb,
        sem_g, sem_g2, sem_w):
    wid = lax.axis_index("s") * NC + lax.axis_index("c")
    base = wid * epw

    pltpu.sync_copy(dst_hbm.at[pl.ds(base, ch)], di.at[0])
    pltpu.sync_copy(src_hbm.at[pl.ds(base, ch)], si.at[0])
    pltpu.async_copy(a_hbm.at[di.at[0]], ra.at[0], sem_g)
    pltpu.async_copy(b_hbm.at[si.at[0]], rb.at[0], sem_g2)
    pltpu.sync_copy(dst_hbm.at[pl.ds(base + ch, ch)], di.at[1])
    pltpu.sync_copy(src_hbm.at[pl.ds(base + ch, ch)], si.at[1])
    pltpu.async_copy(a_hbm.at[di.at[1]], ra.at[1], sem_g)
    pltpu.async_copy(b_hbm.at[si.at[1]], rb.at[1], sem_g2)

    @pl.loop(0, nch)
    def _(j):
      p = lax.rem(j, 3)
      off = base + j * ch

      @pl.when(j + 2 < nch)
      def _():
        r = lax.rem(j + 2, 3)

        @pl.when(j >= 1)
        def _():
          pltpu.make_async_copy(ra.at[r], ga_hbm.at[pl.ds(off, ch)],
                                sem_w).wait()
          pltpu.make_async_copy(rb.at[r], gb_hbm.at[pl.ds(off, ch)],
                                sem_w).wait()

        noff = off + 2 * ch
        pltpu.sync_copy(dst_hbm.at[pl.ds(noff, ch)], di.at[r])
        pltpu.sync_copy(src_hbm.at[pl.ds(noff, ch)], si.at[r])

      pltpu.make_async_copy(a_hbm.at[di.at[p]], ra.at[p], sem_g).wait()
      pltpu.make_async_copy(b_hbm.at[si.at[p]], rb.at[p], sem_g2).wait()

      @pl.when(j + 2 < nch)
      def _():
        r = lax.rem(j + 2, 3)
        pltpu.async_copy(a_hbm.at[di.at[r]], ra.at[r], sem_g)
        pltpu.async_copy(b_hbm.at[si.at[r]], rb.at[r], sem_g2)

      pltpu.async_copy(ra.at[p], ga_hbm.at[pl.ds(off, ch)], sem_w)
      pltpu.async_copy(rb.at[p], gb_hbm.at[pl.ds(off, ch)], sem_w)

    @pl.loop(0, 6)
    def _(j):
      pltpu.make_async_copy(ra.at[0], ga_hbm.at[pl.ds(base, ch)],
                            sem_w).wait()

  return k(A, B, dst, src)


def _sc_scatter_add(m, dst, zerosN):
  E = dst.shape[0]
  N = zerosN.shape[0]
  epw = E // NW
  rpt = N // NSTAGE
  ch = CH if epw % CH == 0 else 40
  nch = epw // ch

  @functools.partial(
      pl.kernel,
      out_type=jax.ShapeDtypeStruct((NC, N, H), jnp.float32),
      mesh=_mesh(),
      scratch_types=[pltpu.VMEM((2, ch), jnp.int32),
                     pltpu.VMEM((2, ch, H), jnp.float32),
                     pltpu.VMEM_SHARED((N, H), jnp.float32),
                     pltpu.SemaphoreType.DMA,
                     pltpu.SemaphoreType.DMA],
  )
  def k(m_hbm, dst_hbm, z_hbm, s_hbm, idx, val, acc_sh, sem_i, sem_s):
    cid = lax.axis_index("c")
    sid = lax.axis_index("s")

    @pl.when(sid < NSTAGE)
    def _():
      pltpu.sync_copy(z_hbm.at[pl.ds(sid * rpt, rpt)],
                      acc_sh.at[pl.ds(sid * rpt, rpt)])

    plsc.subcore_barrier()
    base = (cid * NS + sid) * epw

    pltpu.async_copy(dst_hbm.at[pl.ds(base, ch)], idx.at[0], sem_i)
    pltpu.async_copy(m_hbm.at[pl.ds(base, ch)], val.at[0], sem_i)

    @pl.loop(0, nch)
    def _(j):
      p = lax.rem(j, 2)
      q = 1 - p
      off = base + j * ch

      @pl.when(j + 1 < nch)
      def _():
        @pl.when(j >= 1)
        def _():
          pltpu.make_async_copy(val.at[q], acc_sh.at[idx.at[q]],
                                sem_s).wait()

        noff = off + ch
        pltpu.async_copy(dst_hbm.at[pl.ds(noff, ch)], idx.at[q], sem_i)
        pltpu.async_copy(m_hbm.at[pl.ds(noff, ch)], val.at[q], sem_i)

      pltpu.make_async_copy(dst_hbm.at[pl.ds(off, ch)], idx.at[p],
                            sem_i).wait()
      pltpu.make_async_copy(m_hbm.at[pl.ds(off, ch)], val.at[p],
                            sem_i).wait()
      pltpu.async_copy(val.at[p], acc_sh.at[idx.at[p]], sem_s, add=True)

    pltpu.make_async_copy(val.at[0], acc_sh.at[idx.at[0]], sem_s).wait()
    pltpu.make_async_copy(val.at[1], acc_sh.at[idx.at[1]], sem_s).wait()
    plsc.subcore_barrier()

    @pl.when(sid < NSTAGE)
    def _():
      pltpu.sync_copy(acc_sh.at[pl.ds(sid * rpt, rpt)],
                      s_hbm.at[cid, pl.ds(sid * rpt, rpt)])

  return k(m, dst, zerosN)


def _sc_count(dst, zerosN, ones128):
  E = dst.shape[0]
  N = zerosN.shape[0]
  epw = E // NW
  rpt = N // NSTAGE

  @functools.partial(
      pl.kernel,
      out_type=jax.ShapeDtypeStruct((NC, N, H), jnp.float32),
      mesh=_mesh(),
      scratch_types=[pltpu.VMEM((CH,), jnp.int32),
                     pltpu.VMEM((CH, H), jnp.float32),
                     pltpu.VMEM_SHARED((N, H), jnp.float32)],
  )
  def k(dst_hbm, z_hbm, ones_hbm, c_hbm, idx, val, acc_sh):
    cid = lax.axis_index("c")
    sid = lax.axis_index("s")

    @pl.when(sid < NSTAGE)
    def _():
      pltpu.sync_copy(z_hbm.at[pl.ds(sid * rpt, rpt)],
                      acc_sh.at[pl.ds(sid * rpt, rpt)])

    pltpu.sync_copy(ones_hbm, val)
    plsc.subcore_barrier()
    base = (cid * NS + sid) * epw

    @pl.loop(0, epw // CH)
    def _(j):
      off = base + j * CH
      pltpu.sync_copy(dst_hbm.at[pl.ds(off, CH)], idx)
      pltpu.sync_copy(val, acc_sh.at[idx], add=True)

    plsc.subcore_barrier()

    @pl.when(sid < NSTAGE)
    def _():
      pltpu.sync_copy(acc_sh.at[pl.ds(sid * rpt, rpt)],
                      c_hbm.at[cid, pl.ds(sid * rpt, rpt)])

  return k(dst, zerosN, ones128)


def _full(shape):
  ndim = len(shape)
  return pl.BlockSpec(shape, lambda i: (0,) * ndim)


def _edge_first(ea, encW, encb, w1c, b1, w2, b2):
  E = ea.shape[0]

  def body(ea_ref, ew_ref, eb_ref, wc_ref, b1_ref, w2_ref, b2_ref,
           ef_out, m_out):
    ea_v = ea_ref[...]
    ew = ew_ref[...]
    ef = (ea_v[:, 0:1] * ew[0:1, :] + ea_v[:, 1:2] * ew[1:2, :]
          + ea_v[:, 2:3] * ew[2:3, :] + eb_ref[...])
    pre = jnp.dot(ef, wc_ref[...],
                  preferred_element_type=jnp.float32) + b1_ref[...]
    r = jnp.maximum(pre, 0.0)
    m = jnp.dot(r, w2_ref[...],
                preferred_element_type=jnp.float32) + b2_ref[...]
    m_out[...] = m
    ef_out[...] = ef + m

  return pl.pallas_call(
      body,
      grid=(E // E_BLK,),
      in_specs=[pl.BlockSpec((E_BLK, 3), lambda i: (i, 0)),
                _full((3, H)), _full((H,)), _full((H, H)), _full((H,)),
                _full((H, H)), _full((H,))],
      out_specs=[pl.BlockSpec((E_BLK, H), lambda i: (i, 0)),
                 pl.BlockSpec((E_BLK, H), lambda i: (i, 0))],
      out_shape=[jax.ShapeDtypeStruct((E, H), jnp.float32),
                 jax.ShapeDtypeStruct((E, H), jnp.float32)],
  )(ea, encW, encb, w1c, b1, w2, b2)


def _edge_mid(ga, gb, ef, w1c, b1, w2, b2, want_ef):
  E = ga.shape[0]

  def body_both(ga_ref, gb_ref, ef_ref, wc_ref, b1_ref, w2_ref, b2_ref,
                ef_out, m_out):
    ef_v = ef_ref[...]
    pre = (ga_ref[...] + gb_ref[...] + b1_ref[...]
           + jnp.dot(ef_v, wc_ref[...], preferred_element_type=jnp.float32))
    r = jnp.maximum(pre, 0.0)
    m = jnp.dot(r, w2_ref[...],
                preferred_element_type=jnp.float32) + b2_ref[...]
    m_out[...] = m
    ef_out[...] = ef_v + m

  def body_m(ga_ref, gb_ref, ef_ref, wc_ref, b1_ref, w2_ref, b2_ref, m_out):
    pre = (ga_ref[...] + gb_ref[...] + b1_ref[...]
           + jnp.dot(ef_ref[...], wc_ref[...],
                     preferred_element_type=jnp.float32))
    r = jnp.maximum(pre, 0.0)
    m_out[...] = jnp.dot(r, w2_ref[...],
                         preferred_element_type=jnp.float32) + b2_ref[...]

  eblk = pl.BlockSpec((E_BLK, H), lambda i: (i, 0))
  in_specs = [eblk, eblk, eblk, _full((H, H)), _full((H,)),
              _full((H, H)), _full((H,))]
  if want_ef:
    return pl.pallas_call(
        body_both, grid=(E // E_BLK,), in_specs=in_specs,
        out_specs=[eblk, eblk],
        out_shape=[jax.ShapeDtypeStruct((E, H), jnp.float32),
                   jax.ShapeDtypeStruct((E, H), jnp.float32)],
    )(ga, gb, ef, w1c, b1, w2, b2)
  m = pl.pallas_call(
      body_m, grid=(E // E_BLK,), in_specs=in_specs,
      out_specs=eblk,
      out_shape=jax.ShapeDtypeStruct((E, H), jnp.float32),
  )(ga, gb, ef, w1c, b1, w2, b2)
  return None, m


def _node_mid(x, s0, s1, c0, c1, nw1x, nw1a, nb1, nw2, nb2, w1a_n, w1b_n):
  N = x.shape[0]

  def body(x_ref, s0_ref, s1_ref, c0_ref, c1_ref, w1x_ref,
           w1a_ref, b1_ref, w2_ref, b2_ref, wa_ref, wb_ref,
           x_out, a_out, b_out):
    x_v = x_ref[...]
    cnt = c0_ref[...][:, 0:1] + c1_ref[...][:, 0:1]
    aggr = (s0_ref[...] + s1_ref[...]) / jnp.maximum(cnt, 1.0)
    h = jnp.maximum(
        jnp.dot(x_v, w1x_ref[...], preferred_element_type=jnp.float32)
        + jnp.dot(aggr, w1a_ref[...], preferred_element_type=jnp.float32)
        + b1_ref[...], 0.0)
    xo = x_v + jnp.dot(h, w2_ref[...],
                       preferred_element_type=jnp.float32) + b2_ref[...]
    x_out[...] = xo
    a_out[...] = jnp.dot(xo, wa_ref[...], preferred_element_type=jnp.float32)
    b_out[...] = jnp.dot(xo, wb_ref[...], preferred_element_type=jnp.float32)

  nblk = pl.BlockSpec((N_BLK, H), lambda i: (i, 0))
  return pl.pallas_call(
      body,
      grid=(N // N_BLK,),
      in_specs=[nblk, nblk, nblk, nblk, nblk,
                _full((H, H)), _full((H, H)), _full((H,)), _full((H, H)),
                _full((H,)), _full((H, H)), _full((H, H))],
      out_specs=[nblk, nblk, nblk],
      out_shape=[jax.ShapeDtypeStruct((N, H), jnp.float32),
                 jax.ShapeDtypeStruct((N, H), jnp.float32),
                 jax.ShapeDtypeStruct((N, H), jnp.float32)],
  )(x, s0, s1, c0, c1, nw1x, nw1a, nb1, nw2, nb2, w1a_n, w1b_n)


def _node_last(x, s0, s1, c0, c1, nw1x, nw1a, nb1, nw2, nb2,
               decW128, decb128):
  N = x.shape[0]

  def body(x_ref, s0_ref, s1_ref, c0_ref, c1_ref, w1x_ref,
           w1a_ref, b1_ref, w2_ref, b2_ref, dw_ref, db_ref, o_out):
    x_v = x_ref[...]
    cnt = c0_ref[...][:, 0:1] + c1_ref[...][:, 0:1]
    aggr = (s0_ref[...] + s1_ref[...]) / jnp.maximum(cnt, 1.0)
    h = jnp.maximum(
        jnp.dot(x_v, w1x_ref[...], preferred_element_type=jnp.float32)
        + jnp.dot(aggr, w1a_ref[...], preferred_element_type=jnp.float32)
        + b1_ref[...], 0.0)
    xo = x_v + jnp.dot(h, w2_ref[...],
                       preferred_element_type=jnp.float32) + b2_ref[...]
    o = jnp.dot(xo, dw_ref[...],
                preferred_element_type=jnp.float32) + db_ref[...]
    nrm = jnp.sqrt(jnp.sum(o * o, axis=1, keepdims=True))
    o_out[...] = (o / jnp.maximum(nrm, 1e-12))[:, 0:3]

  nblk = pl.BlockSpec((N_BLK, H), lambda i: (i, 0))
  return pl.pallas_call(
      body,
      grid=(N // N_BLK,),
      in_specs=[nblk, nblk, nblk, nblk, nblk,
                _full((H, H)), _full((H, H)), _full((H,)), _full((H, H)),
                _full((H,)), _full((H, H)), _full((H,))],
      out_specs=pl.BlockSpec((N_BLK, 3), lambda i: (i, 0)),
      out_shape=jax.ShapeDtypeStruct((N, 3), jnp.float32),
  )(x, s0, s1, c0, c1, nw1x, nw1a, nb1, nw2, nb2, decW128, decb128)


def kernel(pos, edge_attr, edge_index, enc_W, enc_b, dec_W, dec_b,
           e_W1, e_b1, e_W2, e_b2, n_W1, n_b1, n_W2, n_b2):
  N = pos.shape[0]
  src = edge_index[0]
  dst = edge_index[1]

  decW128 = jnp.pad(dec_W, ((0, 0), (0, H - 3)))
  decb128 = jnp.pad(dec_b, (0, H - 3))
  zerosN = jnp.zeros((N, H), jnp.float32)
  ones128 = jnp.ones((CH, H), jnp.float32)

  cnt = _sc_count(dst, zerosN, ones128)
  c0, c1 = cnt[0], cnt[1]

  x = zerosN
  A = B = None
  out = None
  ef = None
  for i in range(NLAYERS):
    w1 = e_W1[i]
    w1a, w1b, w1c = w1[0:H], w1[H:2 * H], w1[2 * H:3 * H]
    b1, w2, b2 = e_b1[i], e_W2[i], e_b2[i]
    if i == 0:
      ef, m = _edge_first(edge_attr, enc_W, enc_b, w1c, b1, w2, b2)
    else:
      ga, gb = _sc_gather(A, B, dst, src)
      ef, m = _edge_mid(ga, gb, ef, w1c, b1, w2, b2,
                        want_ef=(i < NLAYERS - 1))
    s = _sc_scatter_add(m, dst, zerosN)
    nw1 = n_W1[i]
    nw1x, nw1a = nw1[0:H], nw1[H:2 * H]
    if i < NLAYERS - 1:
      w1n = e_W1[i + 1]
      x, A, B = _node_mid(x, s[0], s[1], c0, c1, nw1x, nw1a, n_b1[i],
                          n_W2[i], n_b2[i], w1n[0:H], w1n[H:2 * H])
    else:
      out = _node_last(x, s[0], s[1], c0, c1, nw1x, nw1a, n_b1[i],
                       n_W2[i], n_b2[i], decW128, decb128)
  return out


# speedup vs baseline: 1.0511x; 1.0511x over previous
import functools

import jax
import jax.numpy as jnp
from jax import lax
from jax.experimental import pallas as pl
from jax.experimental.pallas import tpu as pltpu
from jax.experimental.pallas import tpu_sc as plsc

H = 128
NLAYERS = 6
NC, NS = 2, 16
NW = NC * NS
CH = 80
E_BLK = 2000
N_BLK = 2000
NSTAGE = 10

@functools.cache
def _mesh():
  return plsc.VectorSubcoreMesh(
      core_axis_name="c", subcore_axis_name="s", num_cores=NC, num_subcores=NS)


def _sc_gather(A, B, dst, src):
  E = dst.shape[0]
  epw = E // NW
  ch = CH if epw % CH == 0 else 40
  nch = epw // ch

  @functools.partial(
      pl.kernel,
      out_type=[jax.ShapeDtypeStruct((E, H), jnp.float32),
                jax.ShapeDtypeStruct((E, H), jnp.float32)],
      mesh=_mesh(),
      scratch_types=[pltpu.VMEM((2, ch), jnp.int32),
                     pltpu.VMEM((2, ch), jnp.int32),
                     pltpu.VMEM((2, ch, H), jnp.float32),
                     pltpu.VMEM((2, ch, H), jnp.float32),
                     pltpu.SemaphoreType.DMA,
                     pltpu.SemaphoreType.DMA,
                     pltpu.SemaphoreType.DMA],
  )
  def k(a_hbm, b_hbm, dst_hbm, src_hbm, ga_hbm, gb_hbm, di, si, ra, rb,
        sem_g, sem_g2, sem_w):
    wid = lax.axis_index("s") * NC + lax.axis_index("c")
    base = wid * epw

    pltpu.sync_copy(dst_hbm.at[pl.ds(base, ch)], di.at[0])
    pltpu.sync_copy(src_hbm.at[pl.ds(base, ch)], si.at[0])
    pltpu.async_copy(a_hbm.at[di.at[0]], ra.at[0], sem_g)
    pltpu.async_copy(b_hbm.at[si.at[0]], rb.at[0], sem_g2)

    @pl.loop(0, nch)
    def _(j):
      p = lax.rem(j, 2)
      q = 1 - p
      off = base + j * ch

      @pl.when(j + 1 < nch)
      def _():
        noff = off + ch
        pltpu.sync_copy(dst_hbm.at[pl.ds(noff, ch)], di.at[q])
        pltpu.sync_copy(src_hbm.at[pl.ds(noff, ch)], si.at[q])

      pltpu.make_async_copy(a_hbm.at[di.at[p]], ra.at[p], sem_g).wait()
      pltpu.make_async_copy(b_hbm.at[si.at[p]], rb.at[p], sem_g2).wait()

      @pl.when(j + 1 < nch)
      def _():
        @pl.when(j >= 1)
        def _():
          pltpu.make_async_copy(ra.at[q], ga_hbm.at[pl.ds(off - ch, ch)],
                                sem_w).wait()
          pltpu.make_async_copy(rb.at[q], gb_hbm.at[pl.ds(off - ch, ch)],
                                sem_w).wait()

        pltpu.async_copy(a_hbm.at[di.at[q]], ra.at[q], sem_g)
        pltpu.async_copy(b_hbm.at[si.at[q]], rb.at[q], sem_g2)

      pltpu.async_copy(ra.at[p], ga_hbm.at[pl.ds(off, ch)], sem_w)
      pltpu.async_copy(rb.at[p], gb_hbm.at[pl.ds(off, ch)], sem_w)

    @pl.loop(0, 4)
    def _(j):
      pltpu.make_async_copy(ra.at[0], ga_hbm.at[pl.ds(base, ch)],
                            sem_w).wait()

  return k(A, B, dst, src)


def _sc_scatter_add(m, dst, zerosN):
  E = dst.shape[0]
  N = zerosN.shape[0]
  epw = E // NW
  rpt = N // NSTAGE
  ch = CH if epw % CH == 0 else 40
  nch = epw // ch

  @functools.partial(
      pl.kernel,
      out_type=jax.ShapeDtypeStruct((NC, N, H), jnp.float32),
      mesh=_mesh(),
      scratch_types=[pltpu.VMEM((2, ch), jnp.int32),
                     pltpu.VMEM((2, ch, H), jnp.float32),
                     pltpu.VMEM_SHARED((N, H), jnp.float32),
                     pltpu.SemaphoreType.DMA,
                     pltpu.SemaphoreType.DMA],
  )
  def k(m_hbm, dst_hbm, z_hbm, s_hbm, idx, val, acc_sh, sem_i, sem_s):
    cid = lax.axis_index("c")
    sid = lax.axis_index("s")

    @pl.when(sid < NSTAGE)
    def _():
      pltpu.sync_copy(z_hbm.at[pl.ds(sid * rpt, rpt)],
                      acc_sh.at[pl.ds(sid * rpt, rpt)])

    plsc.subcore_barrier()
    base = (cid * NS + sid) * epw

    pltpu.async_copy(dst_hbm.at[pl.ds(base, ch)], idx.at[0], sem_i)
    pltpu.async_copy(m_hbm.at[pl.ds(base, ch)], val.at[0], sem_i)

    @pl.loop(0, nch)
    def _(j):
      p = lax.rem(j, 2)
      q = 1 - p
      off = base + j * ch

      @pl.when(j + 1 < nch)
      def _():
        @pl.when(j >= 1)
        def _():
          pltpu.make_async_copy(val.at[q], acc_sh.at[idx.at[q]],
                                sem_s).wait()

        noff = off + ch
        pltpu.async_copy(dst_hbm.at[pl.ds(noff, ch)], idx.at[q], sem_i)
        pltpu.async_copy(m_hbm.at[pl.ds(noff, ch)], val.at[q], sem_i)

      pltpu.make_async_copy(dst_hbm.at[pl.ds(off, ch)], idx.at[p],
                            sem_i).wait()
      pltpu.make_async_copy(m_hbm.at[pl.ds(off, ch)], val.at[p],
                            sem_i).wait()
      pltpu.async_copy(val.at[p], acc_sh.at[idx.at[p]], sem_s, add=True)

    pltpu.make_async_copy(val.at[0], acc_sh.at[idx.at[0]], sem_s).wait()
    pltpu.make_async_copy(val.at[1], acc_sh.at[idx.at[1]], sem_s).wait()
    plsc.subcore_barrier()

    @pl.when(sid < NSTAGE)
    def _():
      pltpu.sync_copy(acc_sh.at[pl.ds(sid * rpt, rpt)],
                      s_hbm.at[cid, pl.ds(sid * rpt, rpt)])

  return k(m, dst, zerosN)


def _sc_count(dst, zerosN, ones128):
  E = dst.shape[0]
  N = zerosN.shape[0]
  epw = E // NW
  rpt = N // NSTAGE

  @functools.partial(
      pl.kernel,
      out_type=jax.ShapeDtypeStruct((NC, N, H), jnp.float32),
      mesh=_mesh(),
      scratch_types=[pltpu.VMEM((CH,), jnp.int32),
                     pltpu.VMEM((CH, H), jnp.float32),
                     pltpu.VMEM_SHARED((N, H), jnp.float32)],
  )
  def k(dst_hbm, z_hbm, ones_hbm, c_hbm, idx, val, acc_sh):
    cid = lax.axis_index("c")
    sid = lax.axis_index("s")

    @pl.when(sid < NSTAGE)
    def _():
      pltpu.sync_copy(z_hbm.at[pl.ds(sid * rpt, rpt)],
                      acc_sh.at[pl.ds(sid * rpt, rpt)])

    pltpu.sync_copy(ones_hbm, val)
    plsc.subcore_barrier()
    base = (cid * NS + sid) * epw

    @pl.loop(0, epw // CH)
    def _(j):
      off = base + j * CH
      pltpu.sync_copy(dst_hbm.at[pl.ds(off, CH)], idx)
      pltpu.sync_copy(val, acc_sh.at[idx], add=True)

    plsc.subcore_barrier()

    @pl.when(sid < NSTAGE)
    def _():
      pltpu.sync_copy(acc_sh.at[pl.ds(sid * rpt, rpt)],
                      c_hbm.at[cid, pl.ds(sid * rpt, rpt)])

  return k(dst, zerosN, ones128)


def _full(shape):
  ndim = len(shape)
  return pl.BlockSpec(shape, lambda i: (0,) * ndim)


def _edge_first(ea, encW, encb, w1c, b1, w2, b2):
  E = ea.shape[0]

  def body(ea_ref, ew_ref, eb_ref, wc_ref, b1_ref, w2_ref, b2_ref,
           ef_out, m_out):
    ea_v = ea_ref[...]
    ew = ew_ref[...]
    ef = (ea_v[:, 0:1] * ew[0:1, :] + ea_v[:, 1:2] * ew[1:2, :]
          + ea_v[:, 2:3] * ew[2:3, :] + eb_ref[...])
    pre = jnp.dot(ef, wc_ref[...],
                  preferred_element_type=jnp.float32) + b1_ref[...]
    r = jnp.maximum(pre, 0.0)
    m = jnp.dot(r, w2_ref[...],
                preferred_element_type=jnp.float32) + b2_ref[...]
    m_out[...] = m
    ef_out[...] = ef + m

  return pl.pallas_call(
      body,
      grid=(E // E_BLK,),
      in_specs=[pl.BlockSpec((E_BLK, 3), lambda i: (i, 0)),
                _full((3, H)), _full((H,)), _full((H, H)), _full((H,)),
                _full((H, H)), _full((H,))],
      out_specs=[pl.BlockSpec((E_BLK, H), lambda i: (i, 0)),
                 pl.BlockSpec((E_BLK, H), lambda i: (i, 0))],
      out_shape=[jax.ShapeDtypeStruct((E, H), jnp.float32),
                 jax.ShapeDtypeStruct((E, H), jnp.float32)],
  )(ea, encW, encb, w1c, b1, w2, b2)


def _edge_mid(ga, gb, ef, w1c, b1, w2, b2, want_ef):
  E = ga.shape[0]

  def body_both(ga_ref, gb_ref, ef_ref, wc_ref, b1_ref, w2_ref, b2_ref,
                ef_out, m_out):
    ef_v = ef_ref[...]
    pre = (ga_ref[...] + gb_ref[...] + b1_ref[...]
           + jnp.dot(ef_v, wc_ref[...], preferred_element_type=jnp.float32))
    r = jnp.maximum(pre, 0.0)
    m = jnp.dot(r, w2_ref[...],
                preferred_element_type=jnp.float32) + b2_ref[...]
    m_out[...] = m
    ef_out[...] = ef_v + m

  def body_m(ga_ref, gb_ref, ef_ref, wc_ref, b1_ref, w2_ref, b2_ref, m_out):
    pre = (ga_ref[...] + gb_ref[...] + b1_ref[...]
           + jnp.dot(ef_ref[...], wc_ref[...],
                     preferred_element_type=jnp.float32))
    r = jnp.maximum(pre, 0.0)
    m_out[...] = jnp.dot(r, w2_ref[...],
                         preferred_element_type=jnp.float32) + b2_ref[...]

  eblk = pl.BlockSpec((E_BLK, H), lambda i: (i, 0))
  in_specs = [eblk, eblk, eblk, _full((H, H)), _full((H,)),
              _full((H, H)), _full((H,))]
  if want_ef:
    return pl.pallas_call(
        body_both, grid=(E // E_BLK,), in_specs=in_specs,
        out_specs=[eblk, eblk],
        out_shape=[jax.ShapeDtypeStruct((E, H), jnp.float32),
                   jax.ShapeDtypeStruct((E, H), jnp.float32)],
    )(ga, gb, ef, w1c, b1, w2, b2)
  m = pl.pallas_call(
      body_m, grid=(E // E_BLK,), in_specs=in_specs,
      out_specs=eblk,
      out_shape=jax.ShapeDtypeStruct((E, H), jnp.float32),
  )(ga, gb, ef, w1c, b1, w2, b2)
  return None, m


def _node_mid(x, s0, s1, c0, c1, nw1x, nw1a, nb1, nw2, nb2, w1a_n, w1b_n):
  N = x.shape[0]

  def body(x_ref, s0_ref, s1_ref, c0_ref, c1_ref, w1x_ref,
           w1a_ref, b1_ref, w2_ref, b2_ref, wa_ref, wb_ref,
           x_out, a_out, b_out):
    x_v = x_ref[...]
    cnt = c0_ref[...][:, 0:1] + c1_ref[...][:, 0:1]
    aggr = (s0_ref[...] + s1_ref[...]) / jnp.maximum(cnt, 1.0)
    h = jnp.maximum(
        jnp.dot(x_v, w1x_ref[...], preferred_element_type=jnp.float32)
        + jnp.dot(aggr, w1a_ref[...], preferred_element_type=jnp.float32)
        + b1_ref[...], 0.0)
    xo = x_v + jnp.dot(h, w2_ref[...],
                       preferred_element_type=jnp.float32) + b2_ref[...]
    x_out[...] = xo
    a_out[...] = jnp.dot(xo, wa_ref[...], preferred_element_type=jnp.float32)
    b_out[...] = jnp.dot(xo, wb_ref[...], preferred_element_type=jnp.float32)

  nblk = pl.BlockSpec((N_BLK, H), lambda i: (i, 0))
  return pl.pallas_call(
      body,
      grid=(N // N_BLK,),
      in_specs=[nblk, nblk, nblk, nblk, nblk,
                _full((H, H)), _full((H, H)), _full((H,)), _full((H, H)),
                _full((H,)), _full((H, H)), _full((H, H))],
      out_specs=[nblk, nblk, nblk],
      out_shape=[jax.ShapeDtypeStruct((N, H), jnp.float32),
                 jax.ShapeDtypeStruct((N, H), jnp.float32),
                 jax.ShapeDtypeStruct((N, H), jnp.float32)],
  )(x, s0, s1, c0, c1, nw1x, nw1a, nb1, nw2, nb2, w1a_n, w1b_n)


def _node_last(x, s0, s1, c0, c1, nw1x, nw1a, nb1, nw2, nb2,
               decW128, decb128):
  N = x.shape[0]

  def body(x_ref, s0_ref, s1_ref, c0_ref, c1_ref, w1x_ref,
           w1a_ref, b1_ref, w2_ref, b2_ref, dw_ref, db_ref, o_out):
    x_v = x_ref[...]
    cnt = c0_ref[...][:, 0:1] + c1_ref[...][:, 0:1]
    aggr = (s0_ref[...] + s1_ref[...]) / jnp.maximum(cnt, 1.0)
    h = jnp.maximum(
        jnp.dot(x_v, w1x_ref[...], preferred_element_type=jnp.float32)
        + jnp.dot(aggr, w1a_ref[...], preferred_element_type=jnp.float32)
        + b1_ref[...], 0.0)
    xo = x_v + jnp.dot(h, w2_ref[...],
                       preferred_element_type=jnp.float32) + b2_ref[...]
    o = jnp.dot(xo, dw_ref[...],
                preferred_element_type=jnp.float32) + db_ref[...]
    nrm = jnp.sqrt(jnp.sum(o * o, axis=1, keepdims=True))
    o_out[...] = (o / jnp.maximum(nrm, 1e-12))[:, 0:3]

  nblk = pl.BlockSpec((N_BLK, H), lambda i: (i, 0))
  return pl.pallas_call(
      body,
      grid=(N // N_BLK,),
      in_specs=[nblk, nblk, nblk, nblk, nblk,
                _full((H, H)), _full((H, H)), _full((H,)), _full((H, H)),
                _full((H,)), _full((H, H)), _full((H,))],
      out_specs=pl.BlockSpec((N_BLK, 3), lambda i: (i, 0)),
      out_shape=jax.ShapeDtypeStruct((N, 3), jnp.float32),
  )(x, s0, s1, c0, c1, nw1x, nw1a, nb1, nw2, nb2, decW128, decb128)


def kernel(pos, edge_attr, edge_index, enc_W, enc_b, dec_W, dec_b,
           e_W1, e_b1, e_W2, e_b2, n_W1, n_b1, n_W2, n_b2):
  N = pos.shape[0]
  src = edge_index[0]
  dst = edge_index[1]

  decW128 = jnp.pad(dec_W, ((0, 0), (0, H - 3)))
  decb128 = jnp.pad(dec_b, (0, H - 3))
  zerosN = jnp.zeros((N, H), jnp.float32)
  ones128 = jnp.ones((CH, H), jnp.float32)

  cnt = _sc_count(dst, zerosN, ones128)
  c0, c1 = cnt[0], cnt[1]

  x = zerosN
  A = B = None
  out = None
  ef = None
  for i in range(NLAYERS):
    w1 = e_W1[i]
    w1a, w1b, w1c = w1[0:H], w1[H:2 * H], w1[2 * H:3 * H]
    b1, w2, b2 = e_b1[i], e_W2[i], e_b2[i]
    if i == 0:
      ef, m = _edge_first(edge_attr, enc_W, enc_b, w1c, b1, w2, b2)
    else:
      ga, gb = _sc_gather(A, B, dst, src)
      ef, m = _edge_mid(ga, gb, ef, w1c, b1, w2, b2,
                        want_ef=(i < NLAYERS - 1))
    s = _sc_scatter_add(m, dst, zerosN)
    nw1 = n_W1[i]
    nw1x, nw1a = nw1[0:H], nw1[H:2 * H]
    if i < NLAYERS - 1:
      w1n = e_W1[i + 1]
      x, A, B = _node_mid(x, s[0], s[1], c0, c1, nw1x, nw1a, n_b1[i],
                          n_W2[i], n_b2[i], w1n[0:H], w1n[H:2 * H])
    else:
      out = _node_last(x, s[0], s[1], c0, c1, nw1x, nw1a, n_b1[i],
                       n_W2[i], n_b2[i], decW128, decb128)
  return out
